# baseline (speedup 1.0000x reference)
import functools

import jax
import jax.numpy as jnp
from jax import lax
from jax.experimental import pallas as pl
from jax.experimental.pallas import tpu as pltpu
from jax.experimental.pallas import tpu_sc as plsc

NC, NS = 2, 16
NW = NC * NS
LANES = 16
NBUF = 4
IDX_CHUNK = 128


@functools.lru_cache(maxsize=None)
def _build(B, L, V, D):
    rows_total = B * L
    seq_per_w = B // NW
    rows_per_w = seq_per_w * L
    n_groups = seq_per_w // NBUF
    assert B % NW == 0 and seq_per_w % NBUF == 0 and D % LANES == 0
    assert L % 8 == 0 and (L - IDX_CHUNK) > 0

    mesh = plsc.VectorSubcoreMesh(
        core_axis_name="c", subcore_axis_name="s", num_cores=NC, num_subcores=NS
    )

    @functools.partial(
        pl.kernel,
        out_type=jax.ShapeDtypeStruct((rows_total, D), jnp.float32),
        mesh=mesh,
        compiler_params=pltpu.CompilerParams(use_tc_tiling_on_sc=False),
        scratch_types=[
            pltpu.VMEM((rows_per_w,), jnp.int32),
            pltpu.VMEM((L, D), jnp.float32),
            pltpu.VMEM((NBUF, L, D), jnp.float32),
            pltpu.SemaphoreType.DMA((NBUF,)),
            pltpu.SemaphoreType.DMA((NBUF,)),
        ],
    )
    def emb(x_hbm, tok_hbm, pos_hbm, out_hbm, idx_v, pos_v, rows_v, gsem, osem):
        wid = lax.axis_index("s") * NC + lax.axis_index("c")
        wbase = pl.multiple_of(wid * rows_per_w, 8)

        pltpu.sync_copy(x_hbm.at[pl.ds(wbase, rows_per_w)], idx_v)
        pltpu.sync_copy(pos_hbm, pos_v)

        def start_gather(s, b):
            off = pl.multiple_of(s * L, 8)
            pltpu.async_copy(
                tok_hbm.at[idx_v.at[pl.ds(off, IDX_CHUNK)]],
                rows_v.at[b, pl.ds(0, IDX_CHUNK)],
                gsem.at[b],
            )
            off2 = pl.multiple_of(s * L + IDX_CHUNK, 8)
            pltpu.async_copy(
                tok_hbm.at[idx_v.at[pl.ds(off2, L - IDX_CHUNK)]],
                rows_v.at[b, pl.ds(IDX_CHUNK, L - IDX_CHUNK)],
                gsem.at[b],
            )

        def wait_gather(b):
            pltpu.make_async_copy(
                out_hbm.at[pl.ds(0, L)], rows_v.at[b], gsem.at[b]
            ).wait()

        def wait_out(b):
            pltpu.make_async_copy(
                rows_v.at[b], out_hbm.at[pl.ds(0, L)], osem.at[b]
            ).wait()

        for b in range(NBUF):
            start_gather(b, b)

        def group_body(g, carry):
            for b in range(NBUF):
                s = g * NBUF + b
                wait_gather(b)

                def row_body(r, c2):
                    for c in range(D // LANES):
                        sl = pl.ds(c * LANES, LANES)
                        rows_v[b, r, sl] = rows_v[b, r, sl] + pos_v[r, sl]
                    return c2

                lax.fori_loop(0, L, row_body, 0, unroll=2)

                orow = pl.multiple_of(wbase + s * L, 8)
                pltpu.async_copy(rows_v.at[b], out_hbm.at[pl.ds(orow, L)], osem.at[b])

                @pl.when(g + 1 < n_groups)
                def _():
                    wait_out(b)
                    start_gather(s + NBUF, b)

            return carry

        lax.fori_loop(0, n_groups, group_body, 0)

        for b in range(NBUF):
            wait_out(b)

    return emb


def kernel(x, token_table, pos_table):
    B, L = x.shape
    V, D = token_table.shape
    x_flat = x.reshape(-1).astype(jnp.int32)
    out = _build(B, L, V, D)(x_flat, token_table, pos_table)
    return out.reshape(B, L, D)
